# baseline (speedup 1.0000x reference)
import functools

import jax
import jax.numpy as jnp
from jax import lax
from jax.experimental import pallas as pl
from jax.experimental.pallas import tpu as pltpu
from jax.experimental.pallas import tpu_sc as plsc

DIM_ = 64
NC_ = 2
NS_ = 16
NW_ = NC_ * NS_
V_ = 1000000
PADD_ = 128

NBLK_ = V_ // PADD_
KMAX_ = (NBLK_ + NW_ - 1) // NW_
TAIL_ = V_ - NBLK_ * PADD_

CHUNK_ = 128
GRP_ = 20
NBUF_ = 10
DEPTH_ = 5


def _transpose_block(slab, pairs, row_vecs, n_cols):
    def p_body(p, carry):
        for h in range(2):
            col = 2 * p + h
            colv = jnp.full((16,), col, dtype=jnp.int32)
            for d16 in range(4):
                v = plsc.load_gather(slab, [row_vecs[d16], colv])
                pairs[p, pl.ds(h * DIM_ + d16 * 16, 16)] = v
        return carry

    lax.fori_loop(0, n_cols // 2, p_body, 0)


@jax.jit
def _relayout_call(t):
    mesh = plsc.VectorSubcoreMesh(core_axis_name="c", subcore_axis_name="s")

    @functools.partial(
        pl.kernel,
        mesh=mesh,
        out_type=jax.ShapeDtypeStruct((V_ // 2, PADD_), jnp.float32),
        scratch_types=[
            pltpu.VMEM((2, DIM_, PADD_), jnp.float32),
            pltpu.VMEM((2, DIM_, PADD_), jnp.float32),
            pltpu.VMEM((DIM_, TAIL_), jnp.float32),
            pltpu.VMEM((TAIL_ // 2, PADD_), jnp.float32),
            pltpu.SemaphoreType.DMA,
            pltpu.SemaphoreType.DMA,
        ],
        compiler_params=pltpu.CompilerParams(use_tc_tiling_on_sc=True,
                                             needs_layout_passes=False),
    )
    def k(t_hbm, out_hbm, slab_v, pairs_v, tslab_v, tpairs_v, lsem, ssem):
        wid = lax.axis_index("s") * NC_ + lax.axis_index("c")
        row_vecs = [d16 * 16 + lax.iota(jnp.int32, 16) for d16 in range(4)]

        def load_slab(bid, buf):
            pltpu.make_async_copy(
                t_hbm.at[:, pl.ds(bid * PADD_, PADD_)], slab_v.at[buf],
                lsem).start()

        def wait_load(buf):
            pltpu.make_async_copy(
                t_hbm.at[:, pl.ds(0, PADD_)], slab_v.at[buf], lsem).wait()

        def store_pairs(bid, buf):
            pltpu.make_async_copy(
                pairs_v.at[buf],
                out_hbm.at[pl.ds(bid * (PADD_ // 2), PADD_ // 2)],
                ssem).start()

        def wait_store(buf):
            pltpu.make_async_copy(
                pairs_v.at[buf], out_hbm.at[pl.ds(0, PADD_ // 2)], ssem).wait()

        load_slab(wid, 0)

        def g_body(g, carry):
            bid = wid + NW_ * g
            buf = g % 2

            @pl.when(bid < NBLK_)
            def _():
                wait_load(buf)

                @pl.when(bid + NW_ < NBLK_)
                def _():
                    load_slab(bid + NW_, 1 - buf)

                @pl.when(g >= 1)
                def _():
                    wait_store(1 - buf)

                _transpose_block(slab_v.at[buf], pairs_v.at[buf], row_vecs,
                                 PADD_)
                store_pairs(bid, buf)
            return carry

        lax.fori_loop(0, KMAX_, g_body, 0)

        n_mine = (NBLK_ - wid + NW_ - 1) // NW_

        @pl.when(n_mine >= 1)
        def _():
            wait_store((n_mine - 1) % 2)

        @pl.when(wid == 0)
        def _():
            pltpu.sync_copy(t_hbm.at[:, pl.ds(NBLK_ * PADD_, TAIL_)], tslab_v)
            _transpose_block(tslab_v, tpairs_v, row_vecs, TAIL_)
            pltpu.sync_copy(
                tpairs_v,
                out_hbm.at[pl.ds(NBLK_ * (PADD_ // 2), TAIL_ // 2)])

    return k(t)


@functools.partial(jax.jit, static_argnames=("n_groups",))
def _gather_call(idx4, table, *, n_groups):
    n_chunks = n_groups * GRP_
    B = NW_ * n_chunks * CHUNK_
    mesh = plsc.VectorSubcoreMesh(core_axis_name="c", subcore_axis_name="s")

    sem_types = [pltpu.SemaphoreType.DMA] * (2 * NBUF_ + 1)

    @functools.partial(
        pl.kernel,
        mesh=mesh,
        out_type=jax.ShapeDtypeStruct((B, PADD_), jnp.float32),
        scratch_types=[
            pltpu.VMEM((2, GRP_, CHUNK_), jnp.int32),
            pltpu.VMEM((NBUF_, CHUNK_, DIM_), jnp.float32),
        ] + sem_types,
        compiler_params=pltpu.CompilerParams(use_tc_tiling_on_sc=False),
    )
    def k(idx_hbm, table_hbm, out_hbm, idx_v, rows_v, *sems):
        gsem = sems[:NBUF_]
        ssem = sems[NBUF_:2 * NBUF_]
        isem = sems[2 * NBUF_:]
        wid = lax.axis_index("s") * NC_ + lax.axis_index("c")
        base = wid * n_chunks * CHUNK_

        def idx_copy(g, gb):
            return pltpu.make_async_copy(idx_hbm.at[wid, g], idx_v.at[gb],
                                         isem[0])

        def start_gather(gb, j, b):
            pltpu.async_copy(table_hbm.at[idx_v.at[gb, j]], rows_v.at[b],
                             gsem[b])

        def wait_gather(b):
            pltpu.make_async_copy(table_hbm.at[idx_v.at[0, 0]], rows_v.at[b],
                                  gsem[b]).wait()

        def start_store(s, b):
            pltpu.async_copy(
                rows_v.at[b],
                out_hbm.at[pl.ds(base + s * CHUNK_, CHUNK_), pl.ds(0, DIM_)],
                ssem[b])

        def wait_store(b):
            pltpu.make_async_copy(
                rows_v.at[b],
                out_hbm.at[pl.ds(base, CHUNK_), pl.ds(0, DIM_)],
                ssem[b]).wait()

        pltpu.sync_copy(idx_hbm.at[wid, 0], idx_v.at[0])
        for j in range(DEPTH_):
            start_gather(0, j, j % NBUF_)

        def group_body(g, carry):
            gb_cur = g % 2
            gb_nxt = (g + 1) % 2
            for j in range(GRP_):
                s = g * GRP_ + j
                b = j % NBUF_

                if j == 0:
                    @pl.when(g + 1 < n_groups)
                    def _():
                        idx_copy(g + 1, gb_nxt).start()

                wait_gather(b)
                start_store(s, b)

                nxt_j = j + DEPTH_
                b2 = nxt_j % NBUF_

                @pl.when(s + DEPTH_ >= NBUF_)
                def _():
                    wait_store(b2)

                if j == GRP_ - DEPTH_:
                    @pl.when(g + 1 < n_groups)
                    def _():
                        idx_copy(g + 1, gb_nxt).wait()

                if nxt_j < GRP_:
                    @pl.when(s + DEPTH_ < n_chunks)
                    def _():
                        start_gather(gb_cur, nxt_j, b2)
                else:
                    @pl.when(s + DEPTH_ < n_chunks)
                    def _():
                        start_gather(gb_nxt, nxt_j - GRP_, b2)
            return carry

        lax.fori_loop(0, n_groups, group_body, 0)

        for i in range(DEPTH_):
            wait_store((n_chunks - DEPTH_ + i) % NBUF_)

    return k(idx4, table)


def kernel(x, emb_weight):
    B = x.shape[0] * x.shape[1]
    n_groups = B // (NW_ * GRP_ * CHUNK_)
    idx4 = x.reshape(NW_, n_groups, GRP_, CHUNK_).astype(jnp.int32)
    table_lin = _relayout_call(emb_weight.T).reshape(V_, DIM_)
    out = _gather_call(idx4, table_lin, n_groups=n_groups)
    return out[:, :DIM_].reshape(x.shape[0], x.shape[1], DIM_)


# speedup vs baseline: 1.4567x; 1.4567x over previous
import functools

import jax
import jax.numpy as jnp
from jax import lax
from jax.experimental import pallas as pl
from jax.experimental.pallas import tpu as pltpu
from jax.experimental.pallas import tpu_sc as plsc

DIM_ = 64
NC_ = 2
NS_ = 16
NW_ = NC_ * NS_
V_ = 1000000
PADD_ = 128

NBLK_ = V_ // PADD_
KMAX_ = (NBLK_ + NW_ - 1) // NW_
TAIL_ = V_ - NBLK_ * PADD_

CHUNK_ = 128
GRP_ = 20
NBUF_ = 10
DEPTH_ = 5


def _transpose_block(slab, pairs, row_vecs, n_cols):
    @plsc.parallel_loop(0, n_cols // 2, unroll=4)
    def p_body(p):
        vals = []
        for h in range(2):
            colv = jnp.full((16,), 2 * p + h, dtype=jnp.int32)
            for d16 in range(4):
                vals.append(plsc.load_gather(slab, [row_vecs[d16], colv]))
        for h in range(2):
            for d16 in range(4):
                pairs[p, pl.ds(h * DIM_ + d16 * 16, 16)] = vals[h * 4 + d16]


@jax.jit
def _relayout_call(t):
    mesh = plsc.VectorSubcoreMesh(core_axis_name="c", subcore_axis_name="s")

    @functools.partial(
        pl.kernel,
        mesh=mesh,
        out_type=jax.ShapeDtypeStruct((V_ // 2, PADD_), jnp.float32),
        scratch_types=[
            pltpu.VMEM((2, DIM_, PADD_), jnp.float32),
            pltpu.VMEM((2, DIM_, PADD_), jnp.float32),
            pltpu.VMEM((DIM_, TAIL_), jnp.float32),
            pltpu.VMEM((TAIL_ // 2, PADD_), jnp.float32),
            pltpu.SemaphoreType.DMA,
            pltpu.SemaphoreType.DMA,
        ],
        compiler_params=pltpu.CompilerParams(use_tc_tiling_on_sc=True,
                                             needs_layout_passes=False),
    )
    def k(t_hbm, out_hbm, slab_v, pairs_v, tslab_v, tpairs_v, lsem, ssem):
        wid = lax.axis_index("s") * NC_ + lax.axis_index("c")
        row_vecs = [d16 * 16 + lax.iota(jnp.int32, 16) for d16 in range(4)]

        def load_slab(bid, buf):
            pltpu.make_async_copy(
                t_hbm.at[:, pl.ds(bid * PADD_, PADD_)], slab_v.at[buf],
                lsem).start()

        def wait_load(buf):
            pltpu.make_async_copy(
                t_hbm.at[:, pl.ds(0, PADD_)], slab_v.at[buf], lsem).wait()

        def store_pairs(bid, buf):
            pltpu.make_async_copy(
                pairs_v.at[buf],
                out_hbm.at[pl.ds(bid * (PADD_ // 2), PADD_ // 2)],
                ssem).start()

        def wait_store(buf):
            pltpu.make_async_copy(
                pairs_v.at[buf], out_hbm.at[pl.ds(0, PADD_ // 2)], ssem).wait()

        load_slab(wid, 0)

        def g_body(g, carry):
            bid = wid + NW_ * g
            buf = g % 2

            @pl.when(bid < NBLK_)
            def _():
                wait_load(buf)

                @pl.when(bid + NW_ < NBLK_)
                def _():
                    load_slab(bid + NW_, 1 - buf)

                @pl.when(g >= 1)
                def _():
                    wait_store(1 - buf)

                _transpose_block(slab_v.at[buf], pairs_v.at[buf], row_vecs,
                                 PADD_)
                store_pairs(bid, buf)
            return carry

        lax.fori_loop(0, KMAX_, g_body, 0)

        n_mine = (NBLK_ - wid + NW_ - 1) // NW_

        @pl.when(n_mine >= 1)
        def _():
            wait_store((n_mine - 1) % 2)

        @pl.when(wid == 0)
        def _():
            pltpu.sync_copy(t_hbm.at[:, pl.ds(NBLK_ * PADD_, TAIL_)], tslab_v)
            _transpose_block(tslab_v, tpairs_v, row_vecs, TAIL_)
            pltpu.sync_copy(
                tpairs_v,
                out_hbm.at[pl.ds(NBLK_ * (PADD_ // 2), TAIL_ // 2)])

    return k(t)


@functools.partial(jax.jit, static_argnames=("n_groups",))
def _gather_call(idx4, table, *, n_groups):
    n_chunks = n_groups * GRP_
    B = NW_ * n_chunks * CHUNK_
    mesh = plsc.VectorSubcoreMesh(core_axis_name="c", subcore_axis_name="s")

    sem_types = [pltpu.SemaphoreType.DMA] * (2 * NBUF_ + 1)

    @functools.partial(
        pl.kernel,
        mesh=mesh,
        out_type=jax.ShapeDtypeStruct((B, PADD_), jnp.float32),
        scratch_types=[
            pltpu.VMEM((2, GRP_, CHUNK_), jnp.int32),
            pltpu.VMEM((NBUF_, CHUNK_, DIM_), jnp.float32),
        ] + sem_types,
        compiler_params=pltpu.CompilerParams(use_tc_tiling_on_sc=False),
    )
    def k(idx_hbm, table_hbm, out_hbm, idx_v, rows_v, *sems):
        gsem = sems[:NBUF_]
        ssem = sems[NBUF_:2 * NBUF_]
        isem = sems[2 * NBUF_:]
        wid = lax.axis_index("s") * NC_ + lax.axis_index("c")
        base = wid * n_chunks * CHUNK_

        def idx_copy(g, gb):
            return pltpu.make_async_copy(idx_hbm.at[wid, g], idx_v.at[gb],
                                         isem[0])

        def start_gather(gb, j, b):
            pltpu.async_copy(table_hbm.at[idx_v.at[gb, j]], rows_v.at[b],
                             gsem[b])

        def wait_gather(b):
            pltpu.make_async_copy(table_hbm.at[idx_v.at[0, 0]], rows_v.at[b],
                                  gsem[b]).wait()

        def start_store(s, b):
            pltpu.async_copy(
                rows_v.at[b],
                out_hbm.at[pl.ds(base + s * CHUNK_, CHUNK_), pl.ds(0, DIM_)],
                ssem[b])

        def wait_store(b):
            pltpu.make_async_copy(
                rows_v.at[b],
                out_hbm.at[pl.ds(base, CHUNK_), pl.ds(0, DIM_)],
                ssem[b]).wait()

        pltpu.sync_copy(idx_hbm.at[wid, 0], idx_v.at[0])
        for j in range(DEPTH_):
            start_gather(0, j, j % NBUF_)

        def group_body(g, carry):
            gb_cur = g % 2
            gb_nxt = (g + 1) % 2
            for j in range(GRP_):
                s = g * GRP_ + j
                b = j % NBUF_

                if j == 0:
                    @pl.when(g + 1 < n_groups)
                    def _():
                        idx_copy(g + 1, gb_nxt).start()

                wait_gather(b)
                start_store(s, b)

                nxt_j = j + DEPTH_
                b2 = nxt_j % NBUF_

                @pl.when(s + DEPTH_ >= NBUF_)
                def _():
                    wait_store(b2)

                if j == GRP_ - DEPTH_:
                    @pl.when(g + 1 < n_groups)
                    def _():
                        idx_copy(g + 1, gb_nxt).wait()

                if nxt_j < GRP_:
                    @pl.when(s + DEPTH_ < n_chunks)
                    def _():
                        start_gather(gb_cur, nxt_j, b2)
                else:
                    @pl.when(s + DEPTH_ < n_chunks)
                    def _():
                        start_gather(gb_nxt, nxt_j - GRP_, b2)
            return carry

        lax.fori_loop(0, n_groups, group_body, 0)

        for i in range(DEPTH_):
            wait_store((n_chunks - DEPTH_ + i) % NBUF_)

    return k(idx4, table)


def kernel(x, emb_weight):
    B = x.shape[0] * x.shape[1]
    n_groups = B // (NW_ * GRP_ * CHUNK_)
    idx4 = x.reshape(NW_, n_groups, GRP_, CHUNK_).astype(jnp.int32)
    table_lin = _relayout_call(emb_weight.T).reshape(V_, DIM_)
    out = _gather_call(idx4, table_lin, n_groups=n_groups)
    return out[:, :DIM_].reshape(x.shape[0], x.shape[1], DIM_)
